# baseline (speedup 1.0000x reference)
import functools

import jax
import jax.numpy as jnp
from jax.experimental import pallas as pl

_NUM_LAYERS = 4
_K = 1024
_D = 64
_BLK = 2048


def _rvq_block(x_ref, emb_ref, emb3_ref, e2_ref, sel_ref, qout_ref, idx_ref,
               loss_ref):
    r = x_ref[:]
    qacc = jnp.zeros_like(r)
    lp = jnp.float32(0.0)
    iota = jax.lax.broadcasted_iota(jnp.int32, (r.shape[0], _K), 1)
    for l in range(_NUM_LAYERS):
        emb = emb_ref[l]
        a = jnp.sum(r * r, axis=1, keepdims=True)
        c = jax.lax.dot_general(
            r, emb, (((1,), (1,)), ((), ())),
            preferred_element_type=jnp.float32,
        )
        dist = (a + e2_ref[l]) - 2.0 * c
        m = jnp.min(dist, axis=1, keepdims=True)
        idx = jnp.min(
            jnp.where(dist == m, iota, jnp.int32(_K)), axis=1, keepdims=True
        )
        onehot = (iota == idx).astype(jnp.bfloat16)
        qp = jax.lax.dot_general(
            onehot, emb3_ref[l], (((1,), (0,)), ((), ())),
            preferred_element_type=jnp.float32,
        )
        q = jax.lax.dot_general(
            qp.astype(jnp.bfloat16), sel_ref[0],
            (((1,), (0,)), ((), ())),
            preferred_element_type=jnp.float32,
        )
        d1 = q - r
        lp = lp + jnp.sum(d1 * d1)
        qst = r + d1
        qacc = qacc + qst
        r = r - qst
        idx_ref[:, l : l + 1] = idx
    qout_ref[:] = qacc
    loss_ref[0] = lp.reshape(1, 1)


@functools.partial(jax.jit, static_argnames=())
def kernel(x, embeddings):
    tokens = x.shape[0] * x.shape[1]
    xf = x.reshape(tokens, _D)
    e2 = jnp.sum(embeddings * embeddings, axis=2).reshape(_NUM_LAYERS, 1, _K)
    mask = jnp.uint32(0xFFFF0000)
    bi = jax.lax.bitcast_convert_type(embeddings, jnp.uint32)
    hi_f = jax.lax.bitcast_convert_type(bi & mask, jnp.float32)
    rem1 = embeddings - hi_f
    b1 = jax.lax.bitcast_convert_type(rem1, jnp.uint32)
    mid_f = jax.lax.bitcast_convert_type(b1 & mask, jnp.float32)
    rem2 = rem1 - mid_f
    e_hi = hi_f.astype(jnp.bfloat16)
    e_mid = mid_f.astype(jnp.bfloat16)
    e_lo = rem2.astype(jnp.bfloat16)
    emb3 = jnp.concatenate([e_hi, e_mid, e_lo], axis=2)
    sel = jnp.tile(jnp.eye(_D, dtype=jnp.bfloat16), (3, 1)).reshape(
        1, 3 * _D, _D
    )
    nblk = tokens // _BLK
    qout, idxs, lparts = pl.pallas_call(
        _rvq_block,
        grid=(nblk,),
        in_specs=[
            pl.BlockSpec((_BLK, _D), lambda i: (i, 0)),
            pl.BlockSpec((_NUM_LAYERS, _K, _D), lambda i: (0, 0, 0)),
            pl.BlockSpec((_NUM_LAYERS, _K, 3 * _D), lambda i: (0, 0, 0)),
            pl.BlockSpec((_NUM_LAYERS, 1, _K), lambda i: (0, 0, 0)),
            pl.BlockSpec((1, 3 * _D, _D), lambda i: (0, 0, 0)),
        ],
        out_specs=[
            pl.BlockSpec((_BLK, _D), lambda i: (i, 0)),
            pl.BlockSpec((_BLK, _NUM_LAYERS), lambda i: (i, 0)),
            pl.BlockSpec((1, 1, 1), lambda i: (i, 0, 0)),
        ],
        out_shape=[
            jax.ShapeDtypeStruct((tokens, _D), jnp.float32),
            jax.ShapeDtypeStruct((tokens, _NUM_LAYERS), jnp.int32),
            jax.ShapeDtypeStruct((nblk, 1, 1), jnp.float32),
        ],
    )(xf, embeddings, emb3, e2, sel)
    quantized_out = qout.reshape(x.shape)
    losses = jnp.sum(lparts) * jnp.float32(0.25 / (tokens * _D))
    all_indices = idxs.reshape(x.shape[0], x.shape[1], _NUM_LAYERS)
    return quantized_out, losses, all_indices


# speedup vs baseline: 1.0396x; 1.0396x over previous
import functools

import jax
import jax.numpy as jnp
from jax.experimental import pallas as pl

_NUM_LAYERS = 4
_K = 1024
_D = 64
_BLK = 2048


def _rvq_block(x_ref, emb_ref, emb3_ref, e2_ref, sel_ref, qout_ref, idx_ref,
               loss_ref):
    r = x_ref[:]
    qacc = jnp.zeros_like(r)
    lp = jnp.float32(0.0)
    iota = jax.lax.broadcasted_iota(jnp.int32, (r.shape[0], _K), 1)
    for l in range(_NUM_LAYERS):
        emb = emb_ref[l]
        a = jnp.sum(r * r, axis=1, keepdims=True)
        c = jax.lax.dot_general(
            r, emb, (((1,), (1,)), ((), ())),
            preferred_element_type=jnp.float32,
        )
        dist = (a + e2_ref[l]) - 2.0 * c
        idx = jnp.argmin(dist, axis=1).astype(jnp.int32)[:, None]
        onehot = (iota == idx).astype(jnp.bfloat16)
        qp = jax.lax.dot_general(
            onehot, emb3_ref[l], (((1,), (0,)), ((), ())),
            preferred_element_type=jnp.float32,
        )
        q = jax.lax.dot_general(
            qp.astype(jnp.bfloat16), sel_ref[0],
            (((1,), (0,)), ((), ())),
            preferred_element_type=jnp.float32,
        )
        d1 = q - r
        lp = lp + jnp.sum(d1 * d1)
        qst = r + d1
        qacc = qacc + qst
        r = r - qst
        idx_ref[:, l : l + 1] = idx
    qout_ref[:] = qacc
    loss_ref[0] = lp.reshape(1, 1)


@functools.partial(jax.jit, static_argnames=())
def kernel(x, embeddings):
    tokens = x.shape[0] * x.shape[1]
    xf = x.reshape(tokens, _D)
    e2 = jnp.sum(embeddings * embeddings, axis=2).reshape(_NUM_LAYERS, 1, _K)
    mask = jnp.uint32(0xFFFF0000)
    bi = jax.lax.bitcast_convert_type(embeddings, jnp.uint32)
    hi_f = jax.lax.bitcast_convert_type(bi & mask, jnp.float32)
    rem1 = embeddings - hi_f
    b1 = jax.lax.bitcast_convert_type(rem1, jnp.uint32)
    mid_f = jax.lax.bitcast_convert_type(b1 & mask, jnp.float32)
    rem2 = rem1 - mid_f
    e_hi = hi_f.astype(jnp.bfloat16)
    e_mid = mid_f.astype(jnp.bfloat16)
    e_lo = rem2.astype(jnp.bfloat16)
    emb3 = jnp.concatenate([e_hi, e_mid, e_lo], axis=2)
    sel = jnp.tile(jnp.eye(_D, dtype=jnp.bfloat16), (3, 1)).reshape(
        1, 3 * _D, _D
    )
    nblk = tokens // _BLK
    qout, idxs, lparts = pl.pallas_call(
        _rvq_block,
        grid=(nblk,),
        in_specs=[
            pl.BlockSpec((_BLK, _D), lambda i: (i, 0)),
            pl.BlockSpec((_NUM_LAYERS, _K, _D), lambda i: (0, 0, 0)),
            pl.BlockSpec((_NUM_LAYERS, _K, 3 * _D), lambda i: (0, 0, 0)),
            pl.BlockSpec((_NUM_LAYERS, 1, _K), lambda i: (0, 0, 0)),
            pl.BlockSpec((1, 3 * _D, _D), lambda i: (0, 0, 0)),
        ],
        out_specs=[
            pl.BlockSpec((_BLK, _D), lambda i: (i, 0)),
            pl.BlockSpec((_BLK, _NUM_LAYERS), lambda i: (i, 0)),
            pl.BlockSpec((1, 1, 1), lambda i: (i, 0, 0)),
        ],
        out_shape=[
            jax.ShapeDtypeStruct((tokens, _D), jnp.float32),
            jax.ShapeDtypeStruct((tokens, _NUM_LAYERS), jnp.int32),
            jax.ShapeDtypeStruct((nblk, 1, 1), jnp.float32),
        ],
    )(xf, embeddings, emb3, e2, sel)
    quantized_out = qout.reshape(x.shape)
    losses = jnp.sum(lparts) * jnp.float32(0.25 / (tokens * _D))
    all_indices = idxs.reshape(x.shape[0], x.shape[1], _NUM_LAYERS)
    return quantized_out, losses, all_indices
